# baseline (speedup 1.0000x reference)
import functools

import numpy as np
import jax
import jax.numpy as jnp
from jax import lax
from jax.experimental import pallas as pl
from jax.experimental.pallas import tpu as pltpu
from jax.experimental.pallas import tpu_sc as plsc

_DIM = 128
_N = 10000
_E = 320000
_NPAD = 10240
_B = 128
_EPT = 20480
_EPAD = 16 * _EPT
_NB = _EPT // _B
_IBB = 80
_AW = 80
_ROWS_PER_TILE = _NPAD // 16

_pp = np.zeros(128, np.int32)
for _c in range(2):
    for _j in range(64):
        _r, _o = _j // 16, _j % 16
        _pp[_c * 64 + _j] = (4 * _c + _o % 4) * 16 + 4 * _r + _o // 4
_PERMQ = _pp


def _interleave32(cols):
    out = np.empty_like(cols)
    for g in range(len(cols) // 32):
        blk = cols[32 * g:32 * g + 32]
        out[32 * g:32 * g + 32:2] = blk[:16]
        out[32 * g + 1:32 * g + 32:2] = blk[16:]
    return out


_PQI0 = _interleave32(_PERMQ[:64])
_PQI1 = _interleave32(_PERMQ[64:])


def _proj_body(x_ref, wq_ref, wkv_ref, bq_ref, bkv_ref, qh_ref, kvh_ref):
    x = x_ref[...]
    qp = jnp.dot(x, wq_ref[...], preferred_element_type=jnp.float32) + bq_ref[...]
    qh_ref[0] = qp[:, :64].astype(jnp.bfloat16)
    qh_ref[1] = qp[:, 64:].astype(jnp.bfloat16)
    kvp = jnp.dot(x, wkv_ref[...], preferred_element_type=jnp.float32) + bkv_ref[...]
    kvh_ref[0] = kvp[:, :128].astype(jnp.bfloat16)
    kvh_ref[1] = kvp[:, 128:].astype(jnp.bfloat16)


def _project(x, wq, wkv, bq2, bkv2):
    return pl.pallas_call(
        _proj_body,
        grid=(5,),
        in_specs=[
            pl.BlockSpec((2000, 128), lambda i: (i, 0)),
            pl.BlockSpec((128, 128), lambda i: (0, 0)),
            pl.BlockSpec((128, 256), lambda i: (0, 0)),
            pl.BlockSpec((1, 128), lambda i: (0, 0)),
            pl.BlockSpec((1, 256), lambda i: (0, 0)),
        ],
        out_specs=[
            pl.BlockSpec((2, 2000, 64), lambda i: (0, i, 0)),
            pl.BlockSpec((2, 2000, 128), lambda i: (0, i, 0)),
        ],
        out_shape=[
            jax.ShapeDtypeStruct((2, _N, 64), jnp.bfloat16),
            jax.ShapeDtypeStruct((2, _N, 128), jnp.bfloat16),
        ],
    )(x, wq, wkv, bq2, bkv2)


def _final_body(acc0_ref, acc1_ref, x_ref, t4_ref, wb0_ref, wb1_ref, wf_ref,
                bf_ref, out_ref):
    a0 = acc0_ref[0]
    a1 = acc1_ref[0]
    t4 = t4_ref[...]
    out0 = a0[:, :64] / (jnp.dot(a0[:, 64:68], t4,
                                 preferred_element_type=jnp.float32) + 1e-16)
    out1 = a1[:, :64] / (jnp.dot(a1[:, 64:68], t4,
                                 preferred_element_type=jnp.float32) + 1e-16)
    out_ref[...] = (
        jnp.dot(out0, wb0_ref[...], preferred_element_type=jnp.float32)
        + jnp.dot(out1, wb1_ref[...], preferred_element_type=jnp.float32)
        + jnp.dot(x_ref[...], wf_ref[...], preferred_element_type=jnp.float32)
        + bf_ref[...]
    )


def _finalize(acc0, acc1, x, t4, wb0, wb1, wf, bf2):
    return pl.pallas_call(
        _final_body,
        grid=(5,),
        in_specs=[
            pl.BlockSpec((1, 2000, _AW), lambda i: (0, i, 0)),
            pl.BlockSpec((1, 2000, _AW), lambda i: (1, i, 0)),
            pl.BlockSpec((2000, 128), lambda i: (i, 0)),
            pl.BlockSpec((4, 64), lambda i: (0, 0)),
            pl.BlockSpec((64, 2), lambda i: (0, 0)),
            pl.BlockSpec((64, 2), lambda i: (0, 0)),
            pl.BlockSpec((128, 2), lambda i: (0, 0)),
            pl.BlockSpec((1, 2), lambda i: (0, 0)),
        ],
        out_specs=pl.BlockSpec((2000, 2), lambda i: (i, 0)),
        out_shape=jax.ShapeDtypeStruct((_N, 2), jnp.float32),
    )(acc0, acc1, x, t4, wb0, wb1, wf, bf2)


def _gather16(x, idx):
    dn = lax.GatherDimensionNumbers(
        offset_dims=(), collapsed_slice_dims=(0,), start_index_map=(0,))
    return lax.gather(x, idx[:, None], dn, slice_sizes=(1,),
                      mode=lax.GatherScatterMode.PROMISE_IN_BOUNDS)


@functools.partial(
    pl.kernel,
    mesh=plsc.VectorSubcoreMesh(core_axis_name="c", subcore_axis_name="s"),
    compiler_params=pltpu.CompilerParams(use_tc_tiling_on_sc=False,
                                         needs_layout_passes=False),
    out_type=jax.ShapeDtypeStruct((2, _NPAD, _AW), jnp.float32),
    scratch_types=[
        pltpu.VMEM((_IBB * _B,), jnp.int32),
        pltpu.VMEM((_IBB * _B,), jnp.int32),
        [pltpu.VMEM((_B,), jnp.int32)] * 2,
        [pltpu.VMEM((_B,), jnp.int32)] * 2,
        [pltpu.VMEM((_B,), jnp.int32)] * 2,
        [pltpu.VMEM((_B, 64), jnp.bfloat16)] * 2,
        [pltpu.VMEM((_B, 128), jnp.bfloat16)] * 2,
        [pltpu.VMEM((_B, _AW), jnp.float32)] * 2,
        pltpu.VMEM_SHARED((_NPAD, _AW), jnp.float32),
        [pltpu.SemaphoreType.DMA] * 2,
        [pltpu.SemaphoreType.DMA] * 2,
        [pltpu.SemaphoreType.DMA] * 2,
    ],
)
def _edge_kernel(qh_hbm, kvh_hbm, dst_hbm, src_hbm, num_hbm,
                 dstbulk, srcbulk, dstb, kvidxb, scatb, qg, kvg, stage, acc,
                 semq, semkv, semsc):
    cid = lax.axis_index("c")
    sid = lax.axis_index("s")

    def zero_stage(i, carry):
        for bb in range(2):
            for col in range(_AW // 16):
                stage[bb][i, pl.ds(col * 16, 16)] = jnp.zeros(
                    (16,), jnp.float32)
        return carry

    lax.fori_loop(0, _B, zero_stage, 0)

    def zero_acc(j, carry):
        pltpu.sync_copy(
            stage[0], acc.at[pl.ds(sid * _ROWS_PER_TILE + j * _B, _B)])
        return carry

    lax.fori_loop(0, _ROWS_PER_TILE // _B, zero_acc, 0)
    plsc.subcore_barrier()

    base = sid * _EPT
    noff = cid * _N

    def load_bulk(blk):
        off = base + blk * (_IBB * _B)
        pltpu.sync_copy(dst_hbm.at[pl.ds(off, _IBB * _B)], dstbulk)
        pltpu.sync_copy(src_hbm.at[pl.ds(off, _IBB * _B)], srcbulk)

    def prep_and_fire(ib, buf):
        boff = lax.rem(ib, _IBB) * _B
        for gg in range(_B // 16):
            o = gg * 16
            dstb[buf][pl.ds(o, 16)] = dstbulk[pl.ds(boff + o, 16)] + noff
            kvidxb[buf][pl.ds(o, 16)] = srcbulk[pl.ds(boff + o, 16)] + noff
        pltpu.async_copy(qh_hbm.at[dstb[buf]], qg[buf], semq[buf])
        pltpu.async_copy(kvh_hbm.at[kvidxb[buf]], kvg[buf], semkv[buf])

    def compute(buf):
        qgb, kvgb, stageb = qg[buf], kvg[buf], stage[buf]

        @plsc.parallel_loop(0, _B, unroll=4)
        def per_edge(e):
            iota = lax.iota(jnp.int32, 16)
            fmt = plsc.PackFormat.INTERLEAVED
            q0, q1 = plsc.unpack(qgb[e, pl.ds(0, 32)], format=fmt)
            q2, q3 = plsc.unpack(qgb[e, pl.ds(32, 32)], format=fmt)
            k0, k1 = plsc.unpack(kvgb[e, pl.ds(0, 32)], format=fmt)
            k2, k3 = plsc.unpack(kvgb[e, pl.ds(32, 32)], format=fmt)
            s = q0 * k0 + q1 * k1 + q2 * k2 + q3 * k3
            t = s + _gather16(s, 8 + (iota & 7))
            u = t + _gather16(t, 4 + (iota & 3))
            w4 = jnp.exp(u * 0.25)
            wdup = _gather16(w4, iota & 3)
            v0, v1 = plsc.unpack(kvgb[e, pl.ds(64, 32)], format=fmt)
            v2, v3 = plsc.unpack(kvgb[e, pl.ds(96, 32)], format=fmt)
            for r, vv in enumerate((v0, v1, v2, v3)):
                stageb[e, pl.ds(16 * r, 16)] = vv * wdup
            stageb[e, pl.ds(64, 16)] = w4

    def fire_scatter(buf):
        for gg in range(_B // 16):
            o = gg * 16
            scatb[buf][pl.ds(o, 16)] = dstb[buf][pl.ds(o, 16)] - noff
        pltpu.async_copy(stage[buf], acc.at[scatb[buf]], semsc[buf],
                         add=True)

    def wait_scatter(buf):
        pltpu.make_async_copy(stage[buf], acc.at[scatb[buf]],
                              semsc[buf]).wait()

    def wait_gathers(buf):
        pltpu.make_async_copy(qh_hbm.at[dstb[buf]], qg[buf], semq[buf]).wait()
        pltpu.make_async_copy(kvh_hbm.at[kvidxb[buf]], kvg[buf],
                              semkv[buf]).wait()

    load_bulk(0)
    prep_and_fire(0, 0)

    def per_pair(it, carry):
        for b in range(2):
            ib = it * 2 + b
            buf = b
            obuf = 1 - b

            @pl.when(jnp.logical_and(ib + 1 < _NB,
                                     lax.rem(ib + 1, _IBB) == 0))
            def _():
                load_bulk((ib + 1) // _IBB)

            @pl.when(ib + 1 < _NB)
            def _():
                prep_and_fire(ib + 1, obuf)

            wait_gathers(buf)

            @pl.when(ib >= 2)
            def _():
                wait_scatter(buf)

            compute(buf)
            fire_scatter(buf)
        return carry

    lax.fori_loop(0, _NB // 2, per_pair, 0)
    wait_scatter(0)
    wait_scatter(1)
    plsc.subcore_barrier()
    pltpu.sync_copy(
        acc.at[pl.ds(sid * _ROWS_PER_TILE, _ROWS_PER_TILE)],
        num_hbm.at[cid, pl.ds(sid * _ROWS_PER_TILE, _ROWS_PER_TILE)])


def kernel(nodes, edge_index, Wq, bq, Wk, bk, Wv, bv, Wskip, bskip, Wb, bb):
    nodes = nodes.astype(jnp.float32)
    src = edge_index[0].astype(jnp.int32)
    dst = edge_index[1].astype(jnp.int32)
    permq = jnp.asarray(_PERMQ)
    p0, p1 = permq[:64], permq[64:]
    pi0, pi1 = jnp.asarray(_PQI0), jnp.asarray(_PQI1)

    wq_p = jnp.concatenate([Wq[:, pi0], Wq[:, pi1]], axis=1)
    wkv_p = jnp.concatenate(
        [Wk[:, pi0], Wv[:, pi0], Wk[:, pi1], Wv[:, pi1]], axis=1)
    bq_p = jnp.concatenate([bq[pi0], bq[pi1]])[None, :]
    bkv_p = jnp.concatenate([bk[pi0], bv[pi0], bk[pi1], bv[pi1]])[None, :]
    qh, kvh = _project(nodes, wq_p, wkv_p, bq_p, bkv_p)

    pad = _EPAD - _E
    dst = jnp.concatenate([dst, jnp.full((pad,), _N, jnp.int32)])
    src = jnp.concatenate([src, jnp.zeros((pad,), jnp.int32)])
    num = _edge_kernel(qh.reshape(2 * _N, 64), kvh.reshape(2 * _N, 128),
                       dst, src)

    t4 = (jnp.arange(64)[None, :] % 4 == jnp.arange(4)[:, None]).astype(
        jnp.float32)
    wb0 = Wb[p0, :]
    wb1 = Wb[p1, :]
    wf = Wskip @ Wb
    bf = (bskip @ Wb + bb)[None, :]
    logits = _finalize(num, num, nodes, t4, wb0, wb1, wf, bf)
    return (logits[:, 0], logits[:, 1])


# speedup vs baseline: 1.4530x; 1.4530x over previous
import functools

import numpy as np
import jax
import jax.numpy as jnp
from jax import lax
from jax.experimental import pallas as pl
from jax.experimental.pallas import tpu as pltpu
from jax.experimental.pallas import tpu_sc as plsc

_DIM = 128
_N = 10000
_E = 320000
_NPAD = 10240
_EPT = _E // 16
_B = 80
_NB = _EPT // _B
_IBB = 125
_AW = 80
_ROWS_PER_TILE = _NPAD // 16

_pp = np.zeros(128, np.int32)
for _c in range(2):
    for _j in range(64):
        _r, _o = _j // 16, _j % 16
        _pp[_c * 64 + _j] = (4 * _c + _o % 4) * 16 + 4 * _r + _o // 4
_PERMQ = _pp


def _interleave32(cols):
    out = np.empty_like(cols)
    for g in range(len(cols) // 32):
        blk = cols[32 * g:32 * g + 32]
        out[32 * g:32 * g + 32:2] = blk[:16]
        out[32 * g + 1:32 * g + 32:2] = blk[16:]
    return out


_PQI0 = _interleave32(_PERMQ[:64])
_PQI1 = _interleave32(_PERMQ[64:])


def _proj_body(x_ref, wq_ref, wkv_ref, bq_ref, bkv_ref, qh_ref, kvh_ref):
    x = x_ref[...]
    qp = jnp.dot(x, wq_ref[...], preferred_element_type=jnp.float32) + bq_ref[...]
    qh_ref[0] = qp[:, :64].astype(jnp.bfloat16)
    qh_ref[1] = qp[:, 64:].astype(jnp.bfloat16)
    kvp = jnp.dot(x, wkv_ref[...], preferred_element_type=jnp.float32) + bkv_ref[...]
    kvh_ref[0] = kvp[:, :128].astype(jnp.bfloat16)
    kvh_ref[1] = kvp[:, 128:].astype(jnp.bfloat16)


def _project(x, wq, wkv, bq2, bkv2):
    return pl.pallas_call(
        _proj_body,
        grid=(5,),
        in_specs=[
            pl.BlockSpec((2000, 128), lambda i: (i, 0)),
            pl.BlockSpec((128, 128), lambda i: (0, 0)),
            pl.BlockSpec((128, 256), lambda i: (0, 0)),
            pl.BlockSpec((1, 128), lambda i: (0, 0)),
            pl.BlockSpec((1, 256), lambda i: (0, 0)),
        ],
        out_specs=[
            pl.BlockSpec((2, 2000, 64), lambda i: (0, i, 0)),
            pl.BlockSpec((2, 2000, 128), lambda i: (0, i, 0)),
        ],
        out_shape=[
            jax.ShapeDtypeStruct((2, _N, 64), jnp.bfloat16),
            jax.ShapeDtypeStruct((2, _N, 128), jnp.bfloat16),
        ],
    )(x, wq, wkv, bq2, bkv2)


def _final_body(acc0_ref, acc1_ref, x_ref, t4_ref, wb0_ref, wb1_ref, wf_ref,
                bf_ref, out_ref):
    a0 = acc0_ref[0]
    a1 = acc1_ref[0]
    t4 = t4_ref[...]
    out0 = a0[:, :64] / (jnp.dot(a0[:, 64:68], t4,
                                 preferred_element_type=jnp.float32) + 1e-16)
    out1 = a1[:, :64] / (jnp.dot(a1[:, 64:68], t4,
                                 preferred_element_type=jnp.float32) + 1e-16)
    out_ref[...] = (
        jnp.dot(out0, wb0_ref[...], preferred_element_type=jnp.float32)
        + jnp.dot(out1, wb1_ref[...], preferred_element_type=jnp.float32)
        + jnp.dot(x_ref[...], wf_ref[...], preferred_element_type=jnp.float32)
        + bf_ref[...]
    )


def _finalize(acc0, acc1, x, t4, wb0, wb1, wf, bf2):
    return pl.pallas_call(
        _final_body,
        grid=(5,),
        in_specs=[
            pl.BlockSpec((1, 2000, _AW), lambda i: (0, i, 0)),
            pl.BlockSpec((1, 2000, _AW), lambda i: (1, i, 0)),
            pl.BlockSpec((2000, 128), lambda i: (i, 0)),
            pl.BlockSpec((4, 64), lambda i: (0, 0)),
            pl.BlockSpec((64, 2), lambda i: (0, 0)),
            pl.BlockSpec((64, 2), lambda i: (0, 0)),
            pl.BlockSpec((128, 2), lambda i: (0, 0)),
            pl.BlockSpec((1, 2), lambda i: (0, 0)),
        ],
        out_specs=pl.BlockSpec((2000, 2), lambda i: (i, 0)),
        out_shape=jax.ShapeDtypeStruct((_N, 2), jnp.float32),
    )(acc0, acc1, x, t4, wb0, wb1, wf, bf2)


def _gather16(x, idx):
    dn = lax.GatherDimensionNumbers(
        offset_dims=(), collapsed_slice_dims=(0,), start_index_map=(0,))
    return lax.gather(x, idx[:, None], dn, slice_sizes=(1,),
                      mode=lax.GatherScatterMode.PROMISE_IN_BOUNDS)


@functools.partial(
    pl.kernel,
    mesh=plsc.VectorSubcoreMesh(core_axis_name="c", subcore_axis_name="s"),
    compiler_params=pltpu.CompilerParams(use_tc_tiling_on_sc=False,
                                         needs_layout_passes=False),
    out_type=jax.ShapeDtypeStruct((2, _NPAD, _AW), jnp.float32),
    scratch_types=[
        pltpu.VMEM((_IBB * _B,), jnp.int32),
        pltpu.VMEM((_IBB * _B,), jnp.int32),
        [pltpu.VMEM((_B,), jnp.int32)] * 2,
        [pltpu.VMEM((_B,), jnp.int32)] * 2,
        [pltpu.VMEM((_B,), jnp.int32)] * 2,
        [pltpu.VMEM((_B, 64), jnp.bfloat16)] * 2,
        [pltpu.VMEM((_B, 128), jnp.bfloat16)] * 2,
        [pltpu.VMEM((_B, _AW), jnp.float32)] * 2,
        pltpu.VMEM_SHARED((_NPAD, _AW), jnp.float32),
        [pltpu.SemaphoreType.DMA] * 2,
        [pltpu.SemaphoreType.DMA] * 2,
        [pltpu.SemaphoreType.DMA] * 2,
    ],
)
def _edge_kernel(qh_hbm, kvh_hbm, dst_hbm, src_hbm, num_hbm,
                 dstbulk, srcbulk, dstb, kvidxb, scatb, qg, kvg, stage, acc,
                 semq, semkv, semsc):
    cid = lax.axis_index("c")
    sid = lax.axis_index("s")

    def zero_stage(i, carry):
        for bb in range(2):
            for col in range(_AW // 16):
                stage[bb][i, pl.ds(col * 16, 16)] = jnp.zeros(
                    (16,), jnp.float32)
        return carry

    lax.fori_loop(0, _B, zero_stage, 0)

    def zero_acc(j, carry):
        pltpu.sync_copy(
            stage[0], acc.at[pl.ds(sid * _ROWS_PER_TILE + j * _B, _B)])
        return carry

    lax.fori_loop(0, _ROWS_PER_TILE // _B, zero_acc, 0)
    plsc.subcore_barrier()

    base = sid * _EPT
    noff = cid * _N

    def load_bulk(blk):
        off = base + blk * (_IBB * _B)
        pltpu.sync_copy(dst_hbm.at[pl.ds(off, _IBB * _B)], dstbulk)
        pltpu.sync_copy(src_hbm.at[pl.ds(off, _IBB * _B)], srcbulk)

    def prep_and_fire(ib, buf):
        boff = lax.rem(ib, _IBB) * _B
        for gg in range(_B // 16):
            o = gg * 16
            dstb[buf][pl.ds(o, 16)] = dstbulk[pl.ds(boff + o, 16)] + noff
            kvidxb[buf][pl.ds(o, 16)] = srcbulk[pl.ds(boff + o, 16)] + noff
        pltpu.async_copy(qh_hbm.at[dstb[buf]], qg[buf], semq[buf])
        pltpu.async_copy(kvh_hbm.at[kvidxb[buf]], kvg[buf], semkv[buf])

    def compute(buf):
        qgb, kvgb, stageb = qg[buf], kvg[buf], stage[buf]

        @plsc.parallel_loop(0, _B, unroll=4)
        def per_edge(e):
            iota = lax.iota(jnp.int32, 16)
            fmt = plsc.PackFormat.INTERLEAVED
            q0, q1 = plsc.unpack(qgb[e, pl.ds(0, 32)], format=fmt)
            q2, q3 = plsc.unpack(qgb[e, pl.ds(32, 32)], format=fmt)
            k0, k1 = plsc.unpack(kvgb[e, pl.ds(0, 32)], format=fmt)
            k2, k3 = plsc.unpack(kvgb[e, pl.ds(32, 32)], format=fmt)
            s = q0 * k0 + q1 * k1 + q2 * k2 + q3 * k3
            t = s + _gather16(s, 8 + (iota & 7))
            u = t + _gather16(t, 4 + (iota & 3))
            w4 = jnp.exp(u * 0.25)
            wdup = _gather16(w4, iota & 3)
            v0, v1 = plsc.unpack(kvgb[e, pl.ds(64, 32)], format=fmt)
            v2, v3 = plsc.unpack(kvgb[e, pl.ds(96, 32)], format=fmt)
            for r, vv in enumerate((v0, v1, v2, v3)):
                stageb[e, pl.ds(16 * r, 16)] = vv * wdup
            stageb[e, pl.ds(64, 16)] = w4

    def fire_scatter(buf):
        for gg in range(_B // 16):
            o = gg * 16
            scatb[buf][pl.ds(o, 16)] = dstb[buf][pl.ds(o, 16)] - noff
        pltpu.async_copy(stage[buf], acc.at[scatb[buf]], semsc[buf],
                         add=True)

    def wait_scatter(buf):
        pltpu.make_async_copy(stage[buf], acc.at[scatb[buf]],
                              semsc[buf]).wait()

    def wait_gathers(buf):
        pltpu.make_async_copy(qh_hbm.at[dstb[buf]], qg[buf], semq[buf]).wait()
        pltpu.make_async_copy(kvh_hbm.at[kvidxb[buf]], kvg[buf],
                              semkv[buf]).wait()

    load_bulk(0)
    prep_and_fire(0, 0)

    def per_pair(it, carry):
        for b in range(2):
            ib = it * 2 + b
            buf = b
            obuf = 1 - b

            @pl.when(jnp.logical_and(ib + 1 < _NB,
                                     lax.rem(ib + 1, _IBB) == 0))
            def _():
                load_bulk((ib + 1) // _IBB)

            @pl.when(ib + 1 < _NB)
            def _():
                prep_and_fire(ib + 1, obuf)

            wait_gathers(buf)

            @pl.when(ib >= 2)
            def _():
                wait_scatter(buf)

            compute(buf)
            fire_scatter(buf)
        return carry

    lax.fori_loop(0, _NB // 2, per_pair, 0)
    wait_scatter(0)
    wait_scatter(1)
    plsc.subcore_barrier()
    pltpu.sync_copy(
        acc.at[pl.ds(sid * _ROWS_PER_TILE, _ROWS_PER_TILE)],
        num_hbm.at[cid, pl.ds(sid * _ROWS_PER_TILE, _ROWS_PER_TILE)])


def kernel(nodes, edge_index, Wq, bq, Wk, bk, Wv, bv, Wskip, bskip, Wb, bb):
    nodes = nodes.astype(jnp.float32)
    src = edge_index[0].astype(jnp.int32)
    dst = edge_index[1].astype(jnp.int32)
    permq = jnp.asarray(_PERMQ)
    p0, p1 = permq[:64], permq[64:]
    pi0, pi1 = jnp.asarray(_PQI0), jnp.asarray(_PQI1)

    wq_p = jnp.concatenate([Wq[:, pi0], Wq[:, pi1]], axis=1)
    wkv_p = jnp.concatenate(
        [Wk[:, pi0], Wv[:, pi0], Wk[:, pi1], Wv[:, pi1]], axis=1)
    bq_p = jnp.concatenate([bq[pi0], bq[pi1]])[None, :]
    bkv_p = jnp.concatenate([bk[pi0], bv[pi0], bk[pi1], bv[pi1]])[None, :]
    qh, kvh = _project(nodes, wq_p, wkv_p, bq_p, bkv_p)

    num = _edge_kernel(qh.reshape(2 * _N, 64), kvh.reshape(2 * _N, 128),
                       dst, src)

    t4 = (jnp.arange(64)[None, :] % 4 == jnp.arange(4)[:, None]).astype(
        jnp.float32)
    wb0 = Wb[p0, :]
    wb1 = Wb[p1, :]
    wf = Wskip @ Wb
    bf = (bskip @ Wb + bb)[None, :]
    logits = _finalize(num, num, nodes, t4, wb0, wb1, wf, bf)
    return (logits[:, 0], logits[:, 1])
